# baseline (speedup 1.0000x reference)
import functools

import jax
import jax.numpy as jnp
from jax import lax
from jax.experimental import pallas as pl
from jax.experimental.pallas import tpu as pltpu
from jax.experimental.pallas import tpu_sc as plsc

N = 10000
E = 320000
D_IN = 205
D = 200
DP = 208
DH = 104
G = 64
LAYERS = 2

N_PAD = 10240
NC = 2
NS = 16
W_EDGE = 128
N_PHASE = 4
WPH = 40
N_WIN = N_PHASE * WPH
EPS = N_WIN * W_EDGE
E_PAD = NS * EPS
ROWS_PER_SUB = N_PAD // NS
BLK = 1024


@functools.lru_cache(maxsize=1)
def _make_segment_sum_sc():
    mesh = plsc.VectorSubcoreMesh(
        core_axis_name="c", subcore_axis_name="s", num_cores=NC, num_subcores=NS
    )

    @functools.partial(
        pl.kernel,
        out_type=jax.ShapeDtypeStruct((N_PAD, DP), jnp.float32),
        mesh=mesh,
        scratch_types=[
            pltpu.VMEM_SHARED((N_PAD + 1, DH), jnp.float32),
            pltpu.VMEM((WPH, W_EDGE), jnp.int32),
            pltpu.VMEM((WPH, W_EDGE), jnp.int32),
            pltpu.VMEM((W_EDGE, DH), jnp.float32),
            pltpu.VMEM((W_EDGE, DH), jnp.float32),
            pltpu.SemaphoreType.DMA,
            pltpu.SemaphoreType.DMA,
        ],
        compiler_params=pltpu.CompilerParams(use_tc_tiling_on_sc=False),
    )
    def _segment_sum_sc(m_hbm, src_hbm, dst_hbm, zeros_hbm, out_hbm,
                        acc, dst_q, src_q, rows0, rows1, sem0, sem1):
        c = lax.axis_index("c")
        s = lax.axis_index("s")
        pltpu.sync_copy(zeros_hbm, acc.at[pl.ds(s * ROWS_PER_SUB, ROWS_PER_SUB), :])
        plsc.subcore_barrier()

        for ph in range(N_PHASE):
            pltpu.sync_copy(src_hbm.at[c, s, ph], src_q)
            pltpu.sync_copy(dst_hbm.at[s, ph], dst_q)
            pltpu.async_copy(m_hbm.at[src_q.at[0]], rows0, sem0)

            def body(g, carry):
                pltpu.async_copy(m_hbm.at[src_q.at[2 * g + 1]], rows1, sem1)
                pltpu.make_async_copy(m_hbm.at[src_q.at[0]], rows0, sem0).wait()
                pltpu.sync_copy(rows0, acc.at[dst_q.at[2 * g]], add=True)

                @pl.when(2 * g + 2 < WPH)
                def _():
                    pltpu.async_copy(m_hbm.at[src_q.at[2 * g + 2]], rows0, sem0)

                pltpu.make_async_copy(m_hbm.at[src_q.at[0]], rows1, sem1).wait()
                pltpu.sync_copy(rows1, acc.at[dst_q.at[2 * g + 1]], add=True)
                return carry

            lax.fori_loop(0, WPH // 2, body, 0)

        plsc.subcore_barrier()
        pltpu.sync_copy(
            acc.at[pl.ds(s * ROWS_PER_SUB, ROWS_PER_SUB), :],
            out_hbm.at[pl.ds(s * ROWS_PER_SUB, ROWS_PER_SUB), pl.ds(c * DH, DH)],
        )

    return _segment_sum_sc


def _proj_body(x_ref, w_ref, b_ref, o_ref):
    o_ref[...] = jnp.maximum(x_ref[...] @ w_ref[...] + b_ref[...], 0.0)


def _mm2_body(h_ref, w1_ref, w2_ref, o_ref):
    h = h_ref[...]
    o_ref[0, :, :] = h @ w1_ref[...]
    o_ref[1, :, :] = h @ w2_ref[...]


def _gh_body(h_ref, whr, whz, whn, bhr, bhz, bhn, gr_ref, gz_ref, gn_ref):
    h = h_ref[...]
    gr_ref[...] = h @ whr[...] + bhr[...]
    gz_ref[...] = h @ whz[...] + bhz[...]
    gn_ref[...] = h @ whn[...] + bhn[...]


def _gru_body(a_ref, h_ref, ghr_ref, ghz_ref, ghn_ref, wir, wiz, win,
              bir, biz, bin_, o_ref):
    a = a_ref[...]
    h = h_ref[...]
    r = jax.nn.sigmoid(a @ wir[...] + bir[...] + ghr_ref[...])
    z = jax.nn.sigmoid(a @ wiz[...] + biz[...] + ghz_ref[...])
    n = jnp.tanh(a @ win[...] + bin_[...] + r * ghn_ref[...])
    o_ref[...] = (1.0 - z) * n + z * h


def _pool_fc_body(starts_ref, h_ref, wfc_ref, bfc_ref, o_ref):
    g = pl.program_id(0)
    start = starts_ref[g]
    end = starts_ref[g + 1]
    chunk0 = start // 8
    nchunk = (end + 7) // 8 - chunk0

    def chunk(i, acc):
        base = (chunk0 + i) * 8
        rows = h_ref[pl.ds(base, 8), :]
        rid = base + lax.broadcasted_iota(jnp.int32, (8, 1), 0)
        keep = (rid >= start) & (rid < end)
        rows = jnp.where(keep, jnp.maximum(rows, 0.0), -jnp.inf)
        return jnp.maximum(acc, jnp.max(rows, axis=0, keepdims=True))

    acc0 = jnp.full((1, D), -jnp.inf, dtype=jnp.float32)
    mx = lax.fori_loop(0, nchunk, chunk, acc0)
    o_ref[pl.ds(g, 1), :] = mx @ wfc_ref[...] + bfc_ref[...]


def _full(shape):
    return pl.BlockSpec(shape, lambda i: (0,) * len(shape))


_proj = pl.pallas_call(
    _proj_body,
    grid=(N_PAD // BLK,),
    in_specs=[
        pl.BlockSpec((BLK, D_IN), lambda i: (i, 0)),
        _full((D_IN, D)),
        _full((1, D)),
    ],
    out_specs=pl.BlockSpec((BLK, D), lambda i: (i, 0)),
    out_shape=jax.ShapeDtypeStruct((N_PAD, D), jnp.float32),
)

_mm2 = pl.pallas_call(
    _mm2_body,
    grid=(N_PAD // BLK,),
    in_specs=[
        pl.BlockSpec((BLK, D), lambda i: (i, 0)),
        _full((D, DH)),
        _full((D, DH)),
    ],
    out_specs=pl.BlockSpec((2, BLK, DH), lambda i: (0, i, 0)),
    out_shape=jax.ShapeDtypeStruct((2, N_PAD, DH), jnp.float32),
)

_gh = pl.pallas_call(
    _gh_body,
    grid=(N_PAD // BLK,),
    in_specs=[pl.BlockSpec((BLK, D), lambda i: (i, 0))]
    + [_full((D, D))] * 3
    + [_full((1, D))] * 3,
    out_specs=[pl.BlockSpec((BLK, D), lambda i: (i, 0))] * 3,
    out_shape=[jax.ShapeDtypeStruct((N_PAD, D), jnp.float32)] * 3,
)

_gru = pl.pallas_call(
    _gru_body,
    grid=(N_PAD // BLK,),
    in_specs=[
        pl.BlockSpec((BLK, DP), lambda i: (i, 0)),
    ]
    + [pl.BlockSpec((BLK, D), lambda i: (i, 0))] * 4
    + [_full((DP, D))] * 3
    + [_full((1, D))] * 3,
    out_specs=pl.BlockSpec((BLK, D), lambda i: (i, 0)),
    out_shape=jax.ShapeDtypeStruct((N_PAD, D), jnp.float32),
)

_pool_fc = pl.pallas_call(
    _pool_fc_body,
    grid=(G,),
    in_specs=[
        pl.BlockSpec(memory_space=pltpu.SMEM),
        _full((N_PAD, D)),
        _full((D, 2)),
        _full((1, 2)),
    ],
    out_specs=_full((G, 2)),
    out_shape=jax.ShapeDtypeStruct((G, 2), jnp.float32),
)


def kernel(x, edge_index, batch, W_proj, b_proj, ggc_w, W_ih, W_hh, b_ih, b_hh,
           W_fc, b_fc):
    f32 = jnp.float32
    i32 = jnp.int32
    x_pad = jnp.zeros((N_PAD, D_IN), f32).at[:N].set(x)
    keys = jnp.sort(edge_index[0] << 14 | edge_index[1])
    src = keys >> 14
    dst = keys & jnp.int32(16383)
    pad = E_PAD - E
    src_p = jnp.concatenate([src, jnp.zeros((pad,), i32)])
    dst_p = jnp.concatenate([dst, jnp.full((pad,), N_PAD, i32)])
    src2 = jnp.stack([src_p, src_p + N_PAD]).reshape(NC, NS, N_PHASE, WPH, W_EDGE)
    dst3 = dst_p.reshape(NS, N_PHASE, WPH, W_EDGE)
    zeros_blk = jnp.zeros((ROWS_PER_SUB, DH), f32)

    W_projT = W_proj.T
    W_ihT = W_ih.T
    W_hhT = W_hh.T
    zpad = jnp.zeros((DP - D, D), f32)
    wir, wiz, win = (jnp.concatenate([W_ihT[:, i * D:(i + 1) * D], zpad])
                     for i in range(3))
    whr, whz, whn = W_hhT[:, 0:D], W_hhT[:, D:2 * D], W_hhT[:, 2 * D:3 * D]
    bir, biz, bin_ = b_ih[0:D][None], b_ih[D:2 * D][None], b_ih[2 * D:][None]
    bhr, bhz, bhn = b_hh[0:D][None], b_hh[D:2 * D][None], b_hh[2 * D:][None]

    starts = jnp.searchsorted(batch, jnp.arange(G + 1, dtype=i32)).astype(i32)

    segsum = _make_segment_sum_sc()

    h = _proj(x_pad, W_projT, b_proj[None])
    for l in range(LAYERS):
        wl = jnp.concatenate([ggc_w[l], jnp.zeros((D, DP - D), f32)], axis=1)
        m2 = _mm2(h, wl[:, :DH], wl[:, DH:])
        ghr, ghz, ghn = _gh(h, whr, whz, whn, bhr, bhz, bhn)
        agg = segsum(m2.reshape(NC * N_PAD, DH), src2, dst3, zeros_blk)
        h = _gru(agg, h, ghr, ghz, ghn, wir, wiz, win, bir, biz, bin_)
    out = _pool_fc(starts, h, W_fc.T, b_fc[None])
    return out


# speedup vs baseline: 1.5099x; 1.5099x over previous
import functools

import jax
import jax.numpy as jnp
from jax import lax
from jax.experimental import pallas as pl
from jax.experimental.pallas import tpu as pltpu
from jax.experimental.pallas import tpu_sc as plsc

N = 10000
E = 320000
D_IN = 205
D = 200
DP = 208
DH = 104
G = 64
LAYERS = 2

N_PAD = 10240
NC = 2
NS = 16
W_EDGE = 128
N_PHASE = 4
WPH = 40
N_WIN = N_PHASE * WPH
EPS = N_WIN * W_EDGE
E_PAD = NS * EPS
ROWS_PER_SUB = N_PAD // NS
BLK = 1024


@functools.lru_cache(maxsize=1)
def _make_segment_sum_sc():
    mesh = plsc.VectorSubcoreMesh(
        core_axis_name="c", subcore_axis_name="s", num_cores=NC, num_subcores=NS
    )

    @functools.partial(
        pl.kernel,
        out_type=jax.ShapeDtypeStruct((N_PAD, DP), jnp.float32),
        mesh=mesh,
        scratch_types=[
            pltpu.VMEM_SHARED((N_PAD + 1, DH), jnp.float32),
            pltpu.VMEM((WPH, W_EDGE), jnp.int32),
            pltpu.VMEM((WPH, W_EDGE), jnp.int32),
            [pltpu.VMEM((W_EDGE, DH), jnp.float32)] * 4,
            [pltpu.SemaphoreType.DMA] * 4,
        ],
        compiler_params=pltpu.CompilerParams(use_tc_tiling_on_sc=False),
    )
    def _segment_sum_sc(m_hbm, src_hbm, dst_hbm, zeros_hbm, out_hbm,
                        acc, dst_q, src_q, rows, sems):
        c = lax.axis_index("c")
        s = lax.axis_index("s")
        pltpu.sync_copy(zeros_hbm, acc.at[pl.ds(s * ROWS_PER_SUB, ROWS_PER_SUB), :])
        plsc.subcore_barrier()

        for ph in range(N_PHASE):
            pltpu.sync_copy(src_hbm.at[c, s, ph], src_q)
            pltpu.sync_copy(dst_hbm.at[s, ph], dst_q)
            for b in range(3):
                pltpu.async_copy(m_hbm.at[src_q.at[b]], rows[b], sems[b])

            def body(g, carry):
                w0 = 4 * g
                for b in range(4):
                    nxt = w0 + b + 3

                    @pl.when(nxt < WPH)
                    def _(b=b, nxt=nxt):
                        pltpu.async_copy(
                            m_hbm.at[src_q.at[nxt]], rows[(b + 3) % 4],
                            sems[(b + 3) % 4])

                    pltpu.make_async_copy(
                        m_hbm.at[src_q.at[0]], rows[b], sems[b]).wait()
                    pltpu.sync_copy(rows[b], acc.at[dst_q.at[w0 + b]], add=True)
                return carry

            lax.fori_loop(0, WPH // 4, body, 0)

        plsc.subcore_barrier()
        pltpu.sync_copy(
            acc.at[pl.ds(s * ROWS_PER_SUB, ROWS_PER_SUB), :],
            out_hbm.at[pl.ds(s * ROWS_PER_SUB, ROWS_PER_SUB), pl.ds(c * DH, DH)],
        )

    return _segment_sum_sc


def _proj_body(x_ref, w_ref, b_ref, o_ref):
    o_ref[...] = jnp.maximum(x_ref[...] @ w_ref[...] + b_ref[...], 0.0)


def _mm2_body(h_ref, w1_ref, w2_ref, o_ref):
    h = h_ref[...]
    o_ref[0, :, :] = h @ w1_ref[...]
    o_ref[1, :, :] = h @ w2_ref[...]


def _gh_body(h_ref, whr, whz, whn, bhr, bhz, bhn, gr_ref, gz_ref, gn_ref):
    h = h_ref[...]
    gr_ref[...] = h @ whr[...] + bhr[...]
    gz_ref[...] = h @ whz[...] + bhz[...]
    gn_ref[...] = h @ whn[...] + bhn[...]


def _gru_body(a_ref, h_ref, ghr_ref, ghz_ref, ghn_ref, wir, wiz, win,
              bir, biz, bin_, o_ref):
    a = a_ref[...]
    h = h_ref[...]
    r = jax.nn.sigmoid(a @ wir[...] + bir[...] + ghr_ref[...])
    z = jax.nn.sigmoid(a @ wiz[...] + biz[...] + ghz_ref[...])
    n = jnp.tanh(a @ win[...] + bin_[...] + r * ghn_ref[...])
    o_ref[...] = (1.0 - z) * n + z * h


def _pool_fc_body(starts_ref, h_ref, wfc_ref, bfc_ref, o_ref):
    g = pl.program_id(0)
    start = starts_ref[g]
    end = starts_ref[g + 1]
    chunk0 = start // 8
    nchunk = (end + 7) // 8 - chunk0

    def chunk(i, acc):
        base = (chunk0 + i) * 8
        rows = h_ref[pl.ds(base, 8), :]
        rid = base + lax.broadcasted_iota(jnp.int32, (8, 1), 0)
        keep = (rid >= start) & (rid < end)
        rows = jnp.where(keep, jnp.maximum(rows, 0.0), -jnp.inf)
        return jnp.maximum(acc, jnp.max(rows, axis=0, keepdims=True))

    acc0 = jnp.full((1, D), -jnp.inf, dtype=jnp.float32)
    mx = lax.fori_loop(0, nchunk, chunk, acc0)
    o_ref[pl.ds(g, 1), :] = mx @ wfc_ref[...] + bfc_ref[...]


def _full(shape):
    return pl.BlockSpec(shape, lambda i: (0,) * len(shape))


_proj = pl.pallas_call(
    _proj_body,
    grid=(N_PAD // BLK,),
    in_specs=[
        pl.BlockSpec((BLK, D_IN), lambda i: (i, 0)),
        _full((D_IN, D)),
        _full((1, D)),
    ],
    out_specs=pl.BlockSpec((BLK, D), lambda i: (i, 0)),
    out_shape=jax.ShapeDtypeStruct((N_PAD, D), jnp.float32),
)

_mm2 = pl.pallas_call(
    _mm2_body,
    grid=(N_PAD // BLK,),
    in_specs=[
        pl.BlockSpec((BLK, D), lambda i: (i, 0)),
        _full((D, DH)),
        _full((D, DH)),
    ],
    out_specs=pl.BlockSpec((2, BLK, DH), lambda i: (0, i, 0)),
    out_shape=jax.ShapeDtypeStruct((2, N_PAD, DH), jnp.float32),
)

_gh = pl.pallas_call(
    _gh_body,
    grid=(N_PAD // BLK,),
    in_specs=[pl.BlockSpec((BLK, D), lambda i: (i, 0))]
    + [_full((D, D))] * 3
    + [_full((1, D))] * 3,
    out_specs=[pl.BlockSpec((BLK, D), lambda i: (i, 0))] * 3,
    out_shape=[jax.ShapeDtypeStruct((N_PAD, D), jnp.float32)] * 3,
)

_gru = pl.pallas_call(
    _gru_body,
    grid=(N_PAD // BLK,),
    in_specs=[
        pl.BlockSpec((BLK, DP), lambda i: (i, 0)),
    ]
    + [pl.BlockSpec((BLK, D), lambda i: (i, 0))] * 4
    + [_full((DP, D))] * 3
    + [_full((1, D))] * 3,
    out_specs=pl.BlockSpec((BLK, D), lambda i: (i, 0)),
    out_shape=jax.ShapeDtypeStruct((N_PAD, D), jnp.float32),
)

_pool_fc = pl.pallas_call(
    _pool_fc_body,
    grid=(G,),
    in_specs=[
        pl.BlockSpec(memory_space=pltpu.SMEM),
        _full((N_PAD, D)),
        _full((D, 2)),
        _full((1, 2)),
    ],
    out_specs=_full((G, 2)),
    out_shape=jax.ShapeDtypeStruct((G, 2), jnp.float32),
)


def kernel(x, edge_index, batch, W_proj, b_proj, ggc_w, W_ih, W_hh, b_ih, b_hh,
           W_fc, b_fc):
    f32 = jnp.float32
    i32 = jnp.int32
    x_pad = jnp.zeros((N_PAD, D_IN), f32).at[:N].set(x)
    src = edge_index[0]
    dst = edge_index[1]
    pad = E_PAD - E
    src_p = jnp.concatenate([src, jnp.zeros((pad,), i32)])
    dst_p = jnp.concatenate([dst, jnp.full((pad,), N_PAD, i32)])
    src2 = jnp.stack([src_p, src_p + N_PAD]).reshape(NC, NS, N_PHASE, WPH, W_EDGE)
    dst3 = dst_p.reshape(NS, N_PHASE, WPH, W_EDGE)
    zeros_blk = jnp.zeros((ROWS_PER_SUB, DH), f32)

    W_projT = W_proj.T
    W_ihT = W_ih.T
    W_hhT = W_hh.T
    zpad = jnp.zeros((DP - D, D), f32)
    wir, wiz, win = (jnp.concatenate([W_ihT[:, i * D:(i + 1) * D], zpad])
                     for i in range(3))
    whr, whz, whn = W_hhT[:, 0:D], W_hhT[:, D:2 * D], W_hhT[:, 2 * D:3 * D]
    bir, biz, bin_ = b_ih[0:D][None], b_ih[D:2 * D][None], b_ih[2 * D:][None]
    bhr, bhz, bhn = b_hh[0:D][None], b_hh[D:2 * D][None], b_hh[2 * D:][None]

    starts = jnp.searchsorted(batch, jnp.arange(G + 1, dtype=i32)).astype(i32)

    segsum = _make_segment_sum_sc()

    h = _proj(x_pad, W_projT, b_proj[None])
    for l in range(LAYERS):
        wl = jnp.concatenate([ggc_w[l], jnp.zeros((D, DP - D), f32)], axis=1)
        m2 = _mm2(h, wl[:, :DH], wl[:, DH:])
        ghr, ghz, ghn = _gh(h, whr, whz, whn, bhr, bhz, bhn)
        agg = segsum(m2.reshape(NC * N_PAD, DH), src2, dst3, zeros_blk)
        h = _gru(agg, h, ghr, ghz, ghn, wir, wiz, win, bir, biz, bin_)
    out = _pool_fc(starts, h, W_fc.T, b_fc[None])
    return out
